# baseline (speedup 1.0000x reference)
import functools

import jax
import jax.numpy as jnp
from jax import lax
from jax.experimental import pallas as pl
from jax.experimental.pallas import tpu as pltpu
from jax.experimental.pallas import tpu_sc as plsc

F = 128
L = 16
NW = 32
CHUNK = 80
GROUPS = CHUNK // L

E_ROWS = 320000
N_PAD = 12800
ET_NUM = 16
AT_NUM = 4


def _zero_flat(ref, n):
    z = jnp.zeros((L,), jnp.float32)
    for i in range(n // L):
        ref[pl.ds(i * L, L)] = z


def _stage_rows(flat_ref, stage, rows):
    for r in range(rows):
        for f in range(F // L):
            stage[r, pl.ds(f * L, L)] = flat_ref[pl.ds(r * F + f * L, L)]


def _stream_accum(feat_hbm, ref_hbm, row0, nchunks, tloc,
                  fb, rb, sems, acc_abs, acc_sq, cnt0):
    lane = lax.broadcasted_iota(jnp.int32, (L,), 0)

    def start(c, slot):
        pltpu.async_copy(feat_hbm.at[pl.ds(row0 + c * CHUNK, CHUNK)],
                         fb.at[slot], sems.at[slot])
        pltpu.async_copy(ref_hbm.at[pl.ds(row0 + c * CHUNK, CHUNK)],
                         rb.at[slot], sems.at[slot])

    def wait(c, slot):
        pltpu.make_async_copy(feat_hbm.at[pl.ds(row0 + c * CHUNK, CHUNK)],
                              fb.at[slot], sems.at[slot]).wait()
        pltpu.make_async_copy(ref_hbm.at[pl.ds(row0 + c * CHUNK, CHUNK)],
                              rb.at[slot], sems.at[slot]).wait()

    def process(c, slot, cnt):
        def g_body(g, cnt):
            for rloc in range(L):
                lrow = c * CHUNK + g * L + rloc
                t_vec = plsc.load_gather(tloc, [jnp.full((L,), lrow, jnp.int32)])
                cnt = cnt + jnp.where(lane == t_vec, 1.0, 0.0)
                tbase = t_vec * F
                for f in range(F // L):
                    e = fb[slot, g * L + rloc, pl.ds(f * L, L)]
                    r = rb[slot, g * L + rloc, pl.ds(f * L, L)]
                    d = e - r
                    idx = tbase + (lane + f * L)
                    plsc.addupdate_scatter(acc_abs, [idx], jnp.abs(d))
                    plsc.addupdate_scatter(acc_sq, [idx], d * d)
            return cnt

        return lax.fori_loop(0, GROUPS, g_body, cnt)

    start(0, 0)
    start(1, 1)

    def pair_body(cc, cnt):
        c0 = 2 * cc
        wait(c0, 0)
        cnt = process(c0, 0, cnt)
        pl.when(c0 + 2 < nchunks)(lambda: start(c0 + 2, 0))
        wait(c0 + 1, 1)
        cnt = process(c0 + 1, 1, cnt)
        pl.when(c0 + 3 < nchunks)(lambda: start(c0 + 3, 1))
        return cnt

    cnt = lax.fori_loop(0, nchunks // 2, pair_body, cnt0)
    wait(nchunks - 1, 0)
    return process(nchunks - 1, 0, cnt)


def _broadcast_counts(cnt_vec, cnt_vmem, cntb, rows):
    cnt_vmem[pl.ds(0, L)] = cnt_vec
    for r in range(rows):
        v = plsc.load_gather(cnt_vmem, [jnp.full((L,), r, jnp.int32)])
        for f in range(F // L):
            cntb[r, pl.ds(f * L, L)] = v


def _sc_partials(edge, ref_edge, et, node, ref_node, at):
    mesh = plsc.VectorSubcoreMesh(core_axis_name="c", subcore_axis_name="s")

    @functools.partial(
        pl.kernel,
        out_type=(
            jax.ShapeDtypeStruct((NW, ET_NUM, F), jnp.float32),
            jax.ShapeDtypeStruct((NW, ET_NUM, F), jnp.float32),
            jax.ShapeDtypeStruct((NW, ET_NUM, F), jnp.float32),
            jax.ShapeDtypeStruct((NW, AT_NUM, F), jnp.float32),
            jax.ShapeDtypeStruct((NW, AT_NUM, F), jnp.float32),
            jax.ShapeDtypeStruct((NW, AT_NUM, F), jnp.float32),
        ),
        mesh=mesh,
        compiler_params=pltpu.CompilerParams(needs_layout_passes=False),
        scratch_types=[
            pltpu.VMEM((2, CHUNK, F), jnp.float32),
            pltpu.VMEM((2, CHUNK, F), jnp.float32),
            pltpu.VMEM((E_ROWS // NW,), jnp.int32),
            pltpu.VMEM((N_PAD // NW,), jnp.int32),
            pltpu.VMEM((ET_NUM * F,), jnp.float32),
            pltpu.VMEM((ET_NUM * F,), jnp.float32),
            pltpu.VMEM((8 * F,), jnp.float32),
            pltpu.VMEM((8 * F,), jnp.float32),
            pltpu.VMEM((L,), jnp.float32),
            pltpu.VMEM((ET_NUM, F), jnp.float32),
            pltpu.SemaphoreType.DMA((2,)),
        ],
    )
    def sc(edge_h, refe_h, et_h, node_h, refn_h, at_h,
           out_ea, out_es, out_ec, out_na, out_ns, out_nc,
           fb, rb, tloc_e, tloc_n, acc_ea, acc_es, acc_na, acc_ns,
           cnt_vmem, cntb, sems):
        wid = lax.axis_index("s") * 2 + lax.axis_index("c")

        _zero_flat(acc_ea, ET_NUM * F)
        _zero_flat(acc_es, ET_NUM * F)
        _zero_flat(acc_na, 8 * F)
        _zero_flat(acc_ns, 8 * F)

        e_per_w = E_ROWS // NW
        n_per_w = N_PAD // NW
        pltpu.sync_copy(et_h.at[pl.ds(wid * e_per_w, e_per_w)], tloc_e)
        pltpu.sync_copy(at_h.at[pl.ds(wid * n_per_w, n_per_w)], tloc_n)

        cnt_e = _stream_accum(edge_h, refe_h, wid * e_per_w, e_per_w // CHUNK,
                              tloc_e, fb, rb, sems, acc_ea, acc_es,
                              jnp.zeros((L,), jnp.float32))
        cnt_n = _stream_accum(node_h, refn_h, wid * n_per_w, n_per_w // CHUNK,
                              tloc_n, fb, rb, sems, acc_na, acc_ns,
                              jnp.zeros((L,), jnp.float32))

        _stage_rows(acc_ea, cntb, ET_NUM)
        pltpu.sync_copy(cntb, out_ea.at[wid])
        _stage_rows(acc_es, cntb, ET_NUM)
        pltpu.sync_copy(cntb, out_es.at[wid])
        _broadcast_counts(cnt_e, cnt_vmem, cntb, ET_NUM)
        pltpu.sync_copy(cntb, out_ec.at[wid])
        _stage_rows(acc_na, cntb, AT_NUM)
        pltpu.sync_copy(cntb.at[pl.ds(0, AT_NUM)], out_na.at[wid])
        _stage_rows(acc_ns, cntb, AT_NUM)
        pltpu.sync_copy(cntb.at[pl.ds(0, AT_NUM)], out_ns.at[wid])
        _broadcast_counts(cnt_n, cnt_vmem, cntb, AT_NUM)
        pltpu.sync_copy(cntb.at[pl.ds(0, AT_NUM)], out_nc.at[wid])

    return sc(edge, ref_edge, et, node, ref_node, at)


def _combine_kernel(ea, es, ec, na, ns, nc, nmask, emask, out):
    def half_loss(s_abs, s_sq, cnt, mask):
        present = (cnt > 0.0).astype(jnp.float32)
        sel = present * mask
        denom = jnp.maximum(cnt, 1.0)
        ncnt = jnp.sum(sel)
        term_abs = jnp.sum(sel * s_abs / denom) / ncnt
        term_sq = jnp.sum(sel * s_sq / denom) / ncnt
        return 0.5 * (term_abs + jnp.sqrt(term_sq))

    hop = half_loss(jnp.sum(ea[...], axis=0), jnp.sum(es[...], axis=0),
                    jnp.sum(ec[...], axis=0), emask[...])
    ons = half_loss(jnp.sum(na[...], axis=0), jnp.sum(ns[...], axis=0),
                    jnp.sum(nc[...], axis=0), nmask[...])
    out[0, 0] = 0.5 * (ons + hop)


def kernel(node_features, ref_node_features, edge_features, ref_edge_features,
           atom_type, edge_type, mask_to_nrme, mask_to_erme):
    at = atom_type.astype(jnp.int32)
    et = edge_type.astype(jnp.int32)
    n = node_features.shape[0]
    at_pad = jnp.concatenate([at, jnp.full((N_PAD - n,), AT_NUM, jnp.int32)])
    zpad = jnp.zeros((N_PAD - n, F), jnp.float32)
    nf = jnp.concatenate([node_features, zpad])
    rnf = jnp.concatenate([ref_node_features, zpad])

    ea, es, ec, na, ns, nc = _sc_partials(edge_features, ref_edge_features, et,
                                          nf, rnf, at_pad)

    loss = pl.pallas_call(
        _combine_kernel,
        out_shape=jax.ShapeDtypeStruct((1, 1), jnp.float32),
        out_specs=pl.BlockSpec(memory_space=pltpu.SMEM),
    )(ea, es, ec, na, ns, nc,
      mask_to_nrme.astype(jnp.float32), mask_to_erme.astype(jnp.float32))
    return loss[0, 0]


# speedup vs baseline: 1.4627x; 1.4627x over previous
import functools

import jax
import jax.numpy as jnp
from jax import lax
from jax.experimental import pallas as pl
from jax.experimental.pallas import tpu as pltpu
from jax.experimental.pallas import tpu_sc as plsc

F = 128
L = 16
NW = 32
CHUNK = 80
GROUPS = CHUNK // L

E_ROWS = 320000
N_PAD = 12800
ET_NUM = 16
AT_NUM = 4


def _zero_rows(ref, rows):
    z = jnp.zeros((L,), jnp.float32)
    for r in range(rows):
        for f in range(F // L):
            ref[r, pl.ds(f * L, L)] = z


def _stream_accum(feat_hbm, ref_hbm, ty_hbm, row0, trow0, nchunks,
                  fb, rb, tb, ab, sb, dsems, ssems, acc_abs, acc_sq,
                  region0, cnt0):
    lane = lax.broadcasted_iota(jnp.int32, (L,), 0)

    def dma_start(c, slot):
        pltpu.async_copy(feat_hbm.at[pl.ds(row0 + c * CHUNK, CHUNK)],
                         fb.at[slot], dsems.at[slot])
        pltpu.async_copy(ref_hbm.at[pl.ds(row0 + c * CHUNK, CHUNK)],
                         rb.at[slot], dsems.at[slot])
        pltpu.async_copy(ty_hbm.at[pl.ds(trow0 + c * CHUNK, CHUNK)],
                         tb.at[c % 4], dsems.at[slot])

    def dma_wait(c, slot):
        pltpu.make_async_copy(feat_hbm.at[pl.ds(row0 + c * CHUNK, CHUNK)],
                              fb.at[slot], dsems.at[slot]).wait()
        pltpu.make_async_copy(ref_hbm.at[pl.ds(row0 + c * CHUNK, CHUNK)],
                              rb.at[slot], dsems.at[slot]).wait()
        pltpu.make_async_copy(ty_hbm.at[pl.ds(trow0 + c * CHUNK, CHUNK)],
                              tb.at[c % 4], dsems.at[slot]).wait()

    def scat_start(c, slot):
        pltpu.async_copy(ab.at[slot], acc_abs.at[tb.at[c % 4]],
                         ssems.at[slot], add=True)
        pltpu.async_copy(sb.at[slot], acc_sq.at[tb.at[c % 4]],
                         ssems.at[slot], add=True)

    def scat_wait(c, slot):
        pltpu.make_async_copy(ab.at[slot], acc_abs.at[tb.at[c % 4]],
                              ssems.at[slot]).wait()
        pltpu.make_async_copy(sb.at[slot], acc_sq.at[tb.at[c % 4]],
                              ssems.at[slot]).wait()

    def process(c, slot, cnt):
        def g_body(g, cnt):
            tv = tb[c % 4, pl.ds(g * L, L)]
            tb[c % 4, pl.ds(g * L, L)] = tv + region0
            for t in range(ET_NUM):
                p = plsc.all_reduce_population_count(tv == t)
                cnt = cnt + jnp.where(lane == t, p, 0)
            for rloc in range(L):
                row = g * L + rloc
                for f in range(F // L):
                    e = fb[slot, row, pl.ds(f * L, L)]
                    r = rb[slot, row, pl.ds(f * L, L)]
                    d = e - r
                    ab[slot, row, pl.ds(f * L, L)] = jnp.abs(d)
                    sb[slot, row, pl.ds(f * L, L)] = d * d
            return cnt

        return lax.fori_loop(0, GROUPS, g_body, cnt)

    dma_start(0, 0)
    dma_start(1, 1)

    def pair_body(cc, cnt):
        c0 = 2 * cc
        dma_wait(c0, 0)
        pl.when(cc >= 1)(lambda: scat_wait(c0 - 2, 0))
        cnt = process(c0, 0, cnt)
        scat_start(c0, 0)
        pl.when(c0 + 2 < nchunks)(lambda: dma_start(c0 + 2, 0))
        dma_wait(c0 + 1, 1)
        pl.when(cc >= 1)(lambda: scat_wait(c0 - 1, 1))
        cnt = process(c0 + 1, 1, cnt)
        scat_start(c0 + 1, 1)
        pl.when(c0 + 3 < nchunks)(lambda: dma_start(c0 + 3, 1))
        return cnt

    cnt = lax.fori_loop(0, nchunks // 2, pair_body, cnt0)
    last = nchunks - 1
    dma_wait(last, 0)
    scat_wait(last - 2, 0)
    cnt = process(last, 0, cnt)
    scat_start(last, 0)
    scat_wait(last, 0)
    scat_wait(last - 1, 1)
    return cnt


def _broadcast_counts(cnt_vec, cnt_vmem, cntb, rows):
    cnt_vmem[pl.ds(0, L)] = cnt_vec
    for r in range(rows):
        v = plsc.load_gather(cnt_vmem, [jnp.full((L,), r, jnp.int32)])
        for f in range(F // L):
            cntb[r, pl.ds(f * L, L)] = v


def _sc_partials(edge, ref_edge, et, node, ref_node, at):
    mesh = plsc.VectorSubcoreMesh(core_axis_name="c", subcore_axis_name="s")

    @functools.partial(
        pl.kernel,
        out_type=(
            jax.ShapeDtypeStruct((NW, ET_NUM, F), jnp.float32),
            jax.ShapeDtypeStruct((NW, ET_NUM, F), jnp.float32),
            jax.ShapeDtypeStruct((NW, ET_NUM, F), jnp.float32),
            jax.ShapeDtypeStruct((NW, AT_NUM, F), jnp.float32),
            jax.ShapeDtypeStruct((NW, AT_NUM, F), jnp.float32),
            jax.ShapeDtypeStruct((NW, AT_NUM, F), jnp.float32),
        ),
        mesh=mesh,
        compiler_params=pltpu.CompilerParams(needs_layout_passes=False),
        scratch_types=[
            pltpu.VMEM((2, CHUNK, F), jnp.float32),
            pltpu.VMEM((2, CHUNK, F), jnp.float32),
            pltpu.VMEM((4, CHUNK), jnp.int32),
            pltpu.VMEM((2, CHUNK, F), jnp.float32),
            pltpu.VMEM((2, CHUNK, F), jnp.float32),
            pltpu.VMEM_SHARED((L * ET_NUM, F), jnp.float32),
            pltpu.VMEM_SHARED((L * ET_NUM, F), jnp.float32),
            pltpu.VMEM_SHARED((L * 8, F), jnp.float32),
            pltpu.VMEM_SHARED((L * 8, F), jnp.float32),
            pltpu.VMEM((L,), jnp.float32),
            pltpu.VMEM((ET_NUM, F), jnp.float32),
            pltpu.SemaphoreType.DMA((2,)),
            pltpu.SemaphoreType.DMA((2,)),
        ],
    )
    def sc(edge_h, refe_h, et_h, node_h, refn_h, at_h,
           out_ea, out_es, out_ec, out_na, out_ns, out_nc,
           fb, rb, tb, ab, sb, acc_ea, acc_es, acc_na, acc_ns,
           cnt_vmem, cntb, dsems, ssems):
        sid = lax.axis_index("s")
        wid = sid * 2 + lax.axis_index("c")

        _zero_rows(cntb, ET_NUM)
        pltpu.sync_copy(cntb, acc_ea.at[pl.ds(sid * ET_NUM, ET_NUM)])
        pltpu.sync_copy(cntb, acc_es.at[pl.ds(sid * ET_NUM, ET_NUM)])
        pltpu.sync_copy(cntb.at[pl.ds(0, 8)], acc_na.at[pl.ds(sid * 8, 8)])
        pltpu.sync_copy(cntb.at[pl.ds(0, 8)], acc_ns.at[pl.ds(sid * 8, 8)])

        e_per_w = E_ROWS // NW
        n_per_w = N_PAD // NW

        cnt_e = _stream_accum(edge_h, refe_h, et_h,
                              wid * e_per_w, wid * e_per_w, e_per_w // CHUNK,
                              fb, rb, tb, ab, sb, dsems, ssems,
                              acc_ea, acc_es, sid * ET_NUM,
                              jnp.zeros((L,), jnp.int32))
        cnt_n = _stream_accum(node_h, refn_h, at_h,
                              wid * n_per_w, wid * n_per_w, n_per_w // CHUNK,
                              fb, rb, tb, ab, sb, dsems, ssems,
                              acc_na, acc_ns, sid * 8,
                              jnp.zeros((L,), jnp.int32))

        pltpu.sync_copy(acc_ea.at[pl.ds(sid * ET_NUM, ET_NUM)], out_ea.at[wid])
        pltpu.sync_copy(acc_es.at[pl.ds(sid * ET_NUM, ET_NUM)], out_es.at[wid])
        _broadcast_counts(cnt_e.astype(jnp.float32), cnt_vmem, cntb, ET_NUM)
        pltpu.sync_copy(cntb, out_ec.at[wid])
        pltpu.sync_copy(acc_na.at[pl.ds(sid * 8, AT_NUM)], out_na.at[wid])
        pltpu.sync_copy(acc_ns.at[pl.ds(sid * 8, AT_NUM)], out_ns.at[wid])
        _broadcast_counts(cnt_n.astype(jnp.float32), cnt_vmem, cntb, AT_NUM)
        pltpu.sync_copy(cntb.at[pl.ds(0, AT_NUM)], out_nc.at[wid])

    return sc(edge, ref_edge, et, node, ref_node, at)


def _combine_kernel(ea, es, ec, na, ns, nc, nmask, emask, out):
    def half_loss(s_abs, s_sq, cnt, mask):
        present = (cnt > 0.0).astype(jnp.float32)
        sel = present * mask
        denom = jnp.maximum(cnt, 1.0)
        ncnt = jnp.sum(sel)
        term_abs = jnp.sum(sel * s_abs / denom) / ncnt
        term_sq = jnp.sum(sel * s_sq / denom) / ncnt
        return 0.5 * (term_abs + jnp.sqrt(term_sq))

    hop = half_loss(jnp.sum(ea[...], axis=0), jnp.sum(es[...], axis=0),
                    jnp.sum(ec[...], axis=0), emask[...])
    ons = half_loss(jnp.sum(na[...], axis=0), jnp.sum(ns[...], axis=0),
                    jnp.sum(nc[...], axis=0), nmask[...])
    out[0, 0] = 0.5 * (ons + hop)


def kernel(node_features, ref_node_features, edge_features, ref_edge_features,
           atom_type, edge_type, mask_to_nrme, mask_to_erme):
    at = atom_type.astype(jnp.int32)
    et = edge_type.astype(jnp.int32)
    n = node_features.shape[0]
    at_pad = jnp.concatenate([at, jnp.full((N_PAD - n,), AT_NUM, jnp.int32)])
    zpad = jnp.zeros((N_PAD - n, F), jnp.float32)
    nf = jnp.concatenate([node_features, zpad])
    rnf = jnp.concatenate([ref_node_features, zpad])

    ea, es, ec, na, ns, nc = _sc_partials(edge_features, ref_edge_features, et,
                                          nf, rnf, at_pad)

    loss = pl.pallas_call(
        _combine_kernel,
        out_shape=jax.ShapeDtypeStruct((1, 1), jnp.float32),
        out_specs=pl.BlockSpec(memory_space=pltpu.SMEM),
    )(ea, es, ec, na, ns, nc,
      mask_to_nrme.astype(jnp.float32), mask_to_erme.astype(jnp.float32))
    return loss[0, 0]


# speedup vs baseline: 3.1521x; 2.1549x over previous
import functools

import jax
import jax.numpy as jnp
from jax import lax
from jax.experimental import pallas as pl
from jax.experimental.pallas import tpu as pltpu
from jax.experimental.pallas import tpu_sc as plsc

F = 128
L = 16
NW = 32
CHUNK = 80
GROUPS = CHUNK // L

E_ROWS = 320000
N_PAD = 12800
ET_NUM = 16
AT_NUM = 4

K_SC = 45
E_SC = NW * CHUNK * K_SC
TCB = NW * CHUNK


def _zero_rows(ref, rows):
    z = jnp.zeros((L,), jnp.float32)
    for r in range(rows):
        for f in range(F // L):
            ref[r, pl.ds(f * L, L)] = z


def _stream_accum(feat_hbm, ref_hbm, ty_hbm, row0, trow0, nchunks,
                  fb, rb, tb, ab, sb, dsems, ssems, acc_abs, acc_sq,
                  region0, cnt0):
    lane = lax.broadcasted_iota(jnp.int32, (L,), 0)

    def dma_start(c, slot):
        pltpu.async_copy(feat_hbm.at[pl.ds(row0 + c * CHUNK, CHUNK)],
                         fb.at[slot], dsems.at[slot])
        pltpu.async_copy(ref_hbm.at[pl.ds(row0 + c * CHUNK, CHUNK)],
                         rb.at[slot], dsems.at[slot])
        pltpu.async_copy(ty_hbm.at[pl.ds(trow0 + c * CHUNK, CHUNK)],
                         tb.at[c % 4], dsems.at[slot])

    def dma_wait(c, slot):
        pltpu.make_async_copy(feat_hbm.at[pl.ds(row0 + c * CHUNK, CHUNK)],
                              fb.at[slot], dsems.at[slot]).wait()
        pltpu.make_async_copy(ref_hbm.at[pl.ds(row0 + c * CHUNK, CHUNK)],
                              rb.at[slot], dsems.at[slot]).wait()
        pltpu.make_async_copy(ty_hbm.at[pl.ds(trow0 + c * CHUNK, CHUNK)],
                              tb.at[c % 4], dsems.at[slot]).wait()

    def scat_start(c, slot):
        pltpu.async_copy(ab.at[slot], acc_abs.at[tb.at[c % 4]],
                         ssems.at[slot], add=True)
        pltpu.async_copy(sb.at[slot], acc_sq.at[tb.at[c % 4]],
                         ssems.at[slot], add=True)

    def scat_wait(c, slot):
        pltpu.make_async_copy(ab.at[slot], acc_abs.at[tb.at[c % 4]],
                              ssems.at[slot]).wait()
        pltpu.make_async_copy(sb.at[slot], acc_sq.at[tb.at[c % 4]],
                              ssems.at[slot]).wait()

    def process(c, slot, cnt):
        def g_body(g, cnt):
            tv = tb[c % 4, pl.ds(g * L, L)]
            tb[c % 4, pl.ds(g * L, L)] = tv + region0
            for t in range(ET_NUM):
                p = plsc.all_reduce_population_count(tv == t)
                cnt = cnt + jnp.where(lane == t, p, 0)
            for rloc in range(L):
                row = g * L + rloc
                for f in range(F // L):
                    e = fb[slot, row, pl.ds(f * L, L)]
                    r = rb[slot, row, pl.ds(f * L, L)]
                    d = e - r
                    ab[slot, row, pl.ds(f * L, L)] = jnp.abs(d)
                    sb[slot, row, pl.ds(f * L, L)] = d * d
            return cnt

        return lax.fori_loop(0, GROUPS, g_body, cnt)

    dma_start(0, 0)
    dma_start(1, 1)

    def pair_body(cc, cnt):
        c0 = 2 * cc
        dma_wait(c0, 0)
        pl.when(cc >= 1)(lambda: scat_wait(c0 - 2, 0))
        cnt = process(c0, 0, cnt)
        scat_start(c0, 0)
        pl.when(c0 + 2 < nchunks)(lambda: dma_start(c0 + 2, 0))
        dma_wait(c0 + 1, 1)
        pl.when(cc >= 1)(lambda: scat_wait(c0 - 1, 1))
        cnt = process(c0 + 1, 1, cnt)
        scat_start(c0 + 1, 1)
        pl.when(c0 + 3 < nchunks)(lambda: dma_start(c0 + 3, 1))
        return cnt

    cnt = lax.fori_loop(0, nchunks // 2, pair_body, cnt0)
    last = nchunks - 1
    dma_wait(last, 0)
    scat_wait(last - 2, 0)
    cnt = process(last, 0, cnt)
    scat_start(last, 0)
    scat_wait(last, 0)
    scat_wait(last - 1, 1)
    return cnt


def _broadcast_counts(cnt_vec, cnt_vmem, cntb, rows):
    cnt_vmem[pl.ds(0, L)] = cnt_vec
    for r in range(rows):
        v = plsc.load_gather(cnt_vmem, [jnp.full((L,), r, jnp.int32)])
        for f in range(F // L):
            cntb[r, pl.ds(f * L, L)] = v


def _sc_partials(edge, ref_edge, et, node, ref_node, at):
    mesh = plsc.VectorSubcoreMesh(core_axis_name="c", subcore_axis_name="s")

    @functools.partial(
        pl.kernel,
        out_type=(
            jax.ShapeDtypeStruct((NW, ET_NUM, F), jnp.float32),
            jax.ShapeDtypeStruct((NW, ET_NUM, F), jnp.float32),
            jax.ShapeDtypeStruct((NW, ET_NUM, F), jnp.float32),
            jax.ShapeDtypeStruct((NW, AT_NUM, F), jnp.float32),
            jax.ShapeDtypeStruct((NW, AT_NUM, F), jnp.float32),
            jax.ShapeDtypeStruct((NW, AT_NUM, F), jnp.float32),
        ),
        mesh=mesh,
        compiler_params=pltpu.CompilerParams(needs_layout_passes=False),
        scratch_types=[
            pltpu.VMEM((2, CHUNK, F), jnp.float32),
            pltpu.VMEM((2, CHUNK, F), jnp.float32),
            pltpu.VMEM((4, CHUNK), jnp.int32),
            pltpu.VMEM((2, CHUNK, F), jnp.float32),
            pltpu.VMEM((2, CHUNK, F), jnp.float32),
            pltpu.VMEM_SHARED((L * ET_NUM, F), jnp.float32),
            pltpu.VMEM_SHARED((L * ET_NUM, F), jnp.float32),
            pltpu.VMEM_SHARED((L * 8, F), jnp.float32),
            pltpu.VMEM_SHARED((L * 8, F), jnp.float32),
            pltpu.VMEM((L,), jnp.float32),
            pltpu.VMEM((ET_NUM, F), jnp.float32),
            pltpu.SemaphoreType.DMA((2,)),
            pltpu.SemaphoreType.DMA((2,)),
        ],
    )
    def sc(edge_h, refe_h, et_h, node_h, refn_h, at_h,
           out_ea, out_es, out_ec, out_na, out_ns, out_nc,
           fb, rb, tb, ab, sb, acc_ea, acc_es, acc_na, acc_ns,
           cnt_vmem, cntb, dsems, ssems):
        sid = lax.axis_index("s")
        wid = sid * 2 + lax.axis_index("c")

        _zero_rows(cntb, ET_NUM)
        pltpu.sync_copy(cntb, acc_ea.at[pl.ds(sid * ET_NUM, ET_NUM)])
        pltpu.sync_copy(cntb, acc_es.at[pl.ds(sid * ET_NUM, ET_NUM)])
        pltpu.sync_copy(cntb.at[pl.ds(0, 8)], acc_na.at[pl.ds(sid * 8, 8)])
        pltpu.sync_copy(cntb.at[pl.ds(0, 8)], acc_ns.at[pl.ds(sid * 8, 8)])

        e_per_w = E_SC // NW
        n_per_w = N_PAD // NW

        cnt_e = _stream_accum(edge_h, refe_h, et_h,
                              wid * e_per_w, wid * e_per_w, e_per_w // CHUNK,
                              fb, rb, tb, ab, sb, dsems, ssems,
                              acc_ea, acc_es, sid * ET_NUM,
                              jnp.zeros((L,), jnp.int32))
        cnt_n = _stream_accum(node_h, refn_h, at_h,
                              wid * n_per_w, wid * n_per_w, n_per_w // CHUNK,
                              fb, rb, tb, ab, sb, dsems, ssems,
                              acc_na, acc_ns, sid * 8,
                              jnp.zeros((L,), jnp.int32))

        pltpu.sync_copy(acc_ea.at[pl.ds(sid * ET_NUM, ET_NUM)], out_ea.at[wid])
        pltpu.sync_copy(acc_es.at[pl.ds(sid * ET_NUM, ET_NUM)], out_es.at[wid])
        _broadcast_counts(cnt_e.astype(jnp.float32), cnt_vmem, cntb, ET_NUM)
        pltpu.sync_copy(cntb, out_ec.at[wid])
        pltpu.sync_copy(acc_na.at[pl.ds(sid * 8, AT_NUM)], out_na.at[wid])
        pltpu.sync_copy(acc_ns.at[pl.ds(sid * 8, AT_NUM)], out_ns.at[wid])
        _broadcast_counts(cnt_n.astype(jnp.float32), cnt_vmem, cntb, AT_NUM)
        pltpu.sync_copy(cntb.at[pl.ds(0, AT_NUM)], out_nc.at[wid])

    return sc(edge, ref_edge, et, node, ref_node, at)


def _tc_edge_kernel(ft, rt, ty, oa, os_, oc):
    i = pl.program_id(0)
    d = ft[...] - rt[...]
    a = jnp.abs(d)
    s = d * d
    onehot = (lax.broadcasted_iota(jnp.int32, (ET_NUM, TCB), 0)
              == ty[...]).astype(jnp.float32)
    dims = (((1,), (0,)), ((), ()))
    pa = lax.dot_general(onehot, a, dims, preferred_element_type=jnp.float32)
    ps = lax.dot_general(onehot, s, dims, preferred_element_type=jnp.float32)
    pc = lax.dot_general(onehot, jnp.ones((TCB, F), jnp.float32), dims,
                         preferred_element_type=jnp.float32)

    @pl.when(i == 0)
    def _init():
        oa[...] = jnp.zeros_like(oa)
        os_[...] = jnp.zeros_like(os_)
        oc[...] = jnp.zeros_like(oc)

    oa[...] += pa
    os_[...] += ps
    oc[...] += pc


def _tc_partials(edge, ref_edge, etf):
    nblocks = (E_ROWS - E_SC) // TCB
    blk0 = E_SC // TCB
    return pl.pallas_call(
        _tc_edge_kernel,
        grid=(nblocks,),
        in_specs=[
            pl.BlockSpec((TCB, F), lambda i: (blk0 + i, 0)),
            pl.BlockSpec((TCB, F), lambda i: (blk0 + i, 0)),
            pl.BlockSpec((1, TCB), lambda i: (0, blk0 + i)),
        ],
        out_specs=[
            pl.BlockSpec((ET_NUM, F), lambda i: (0, 0)),
            pl.BlockSpec((ET_NUM, F), lambda i: (0, 0)),
            pl.BlockSpec((ET_NUM, F), lambda i: (0, 0)),
        ],
        out_shape=[
            jax.ShapeDtypeStruct((ET_NUM, F), jnp.float32),
            jax.ShapeDtypeStruct((ET_NUM, F), jnp.float32),
            jax.ShapeDtypeStruct((ET_NUM, F), jnp.float32),
        ],
    )(edge, ref_edge, etf)


def _combine_kernel(ea, es, ec, na, ns, nc, ta, ts, tcn, nmask, emask, out):
    def half_loss(s_abs, s_sq, cnt, mask):
        present = (cnt > 0.0).astype(jnp.float32)
        sel = present * mask
        denom = jnp.maximum(cnt, 1.0)
        ncnt = jnp.sum(sel)
        term_abs = jnp.sum(sel * s_abs / denom) / ncnt
        term_sq = jnp.sum(sel * s_sq / denom) / ncnt
        return 0.5 * (term_abs + jnp.sqrt(term_sq))

    hop = half_loss(jnp.sum(ea[...], axis=0) + ta[...],
                    jnp.sum(es[...], axis=0) + ts[...],
                    jnp.sum(ec[...], axis=0) + tcn[...], emask[...])
    ons = half_loss(jnp.sum(na[...], axis=0), jnp.sum(ns[...], axis=0),
                    jnp.sum(nc[...], axis=0), nmask[...])
    out[0, 0] = 0.5 * (ons + hop)


def kernel(node_features, ref_node_features, edge_features, ref_edge_features,
           atom_type, edge_type, mask_to_nrme, mask_to_erme):
    at = atom_type.astype(jnp.int32)
    et = edge_type.astype(jnp.int32)
    n = node_features.shape[0]
    at_pad = jnp.concatenate([at, jnp.full((N_PAD - n,), AT_NUM, jnp.int32)])
    zpad = jnp.zeros((N_PAD - n, F), jnp.float32)
    nf = jnp.concatenate([node_features, zpad])
    rnf = jnp.concatenate([ref_node_features, zpad])

    ea, es, ec, na, ns, nc = _sc_partials(edge_features, ref_edge_features, et,
                                          nf, rnf, at_pad)
    etf = et.reshape(1, E_ROWS)
    ta, ts, tcn = _tc_partials(edge_features, ref_edge_features, etf)

    loss = pl.pallas_call(
        _combine_kernel,
        out_shape=jax.ShapeDtypeStruct((1, 1), jnp.float32),
        out_specs=pl.BlockSpec(memory_space=pltpu.SMEM),
    )(ea, es, ec, na, ns, nc, ta, ts, tcn,
      mask_to_nrme.astype(jnp.float32), mask_to_erme.astype(jnp.float32))
    return loss[0, 0]


# speedup vs baseline: 3.7301x; 1.1834x over previous
import functools

import jax
import jax.numpy as jnp
from jax import lax
from jax.experimental import pallas as pl
from jax.experimental.pallas import tpu as pltpu
from jax.experimental.pallas import tpu_sc as plsc

F = 128
L = 16
NW = 32
CHUNK = 80
GROUPS = CHUNK // L

E_ROWS = 320000
N_PAD = 12800
ET_NUM = 16
AT_NUM = 4

K_SC = 35
E_SC = NW * CHUNK * K_SC
TCB = NW * CHUNK


def _zero_rows(ref, rows):
    z = jnp.zeros((L,), jnp.float32)
    for r in range(rows):
        for f in range(F // L):
            ref[r, pl.ds(f * L, L)] = z


def _stream_accum(feat_hbm, ref_hbm, ty_hbm, row0, trow0, nchunks,
                  fb, rb, tb, ab, sb, dsems, ssems, acc_abs, acc_sq,
                  region0, cnt0):
    lane = lax.broadcasted_iota(jnp.int32, (L,), 0)

    def dma_start(c, slot):
        pltpu.async_copy(feat_hbm.at[pl.ds(row0 + c * CHUNK, CHUNK)],
                         fb.at[slot], dsems.at[slot])
        pltpu.async_copy(ref_hbm.at[pl.ds(row0 + c * CHUNK, CHUNK)],
                         rb.at[slot], dsems.at[slot])
        pltpu.async_copy(ty_hbm.at[pl.ds(trow0 + c * CHUNK, CHUNK)],
                         tb.at[c % 4], dsems.at[slot])

    def dma_wait(c, slot):
        pltpu.make_async_copy(feat_hbm.at[pl.ds(row0 + c * CHUNK, CHUNK)],
                              fb.at[slot], dsems.at[slot]).wait()
        pltpu.make_async_copy(ref_hbm.at[pl.ds(row0 + c * CHUNK, CHUNK)],
                              rb.at[slot], dsems.at[slot]).wait()
        pltpu.make_async_copy(ty_hbm.at[pl.ds(trow0 + c * CHUNK, CHUNK)],
                              tb.at[c % 4], dsems.at[slot]).wait()

    def scat_start(c, slot):
        pltpu.async_copy(ab.at[slot], acc_abs.at[tb.at[c % 4]],
                         ssems.at[slot], add=True)
        pltpu.async_copy(sb.at[slot], acc_sq.at[tb.at[c % 4]],
                         ssems.at[slot], add=True)

    def scat_wait(c, slot):
        pltpu.make_async_copy(ab.at[slot], acc_abs.at[tb.at[c % 4]],
                              ssems.at[slot]).wait()
        pltpu.make_async_copy(sb.at[slot], acc_sq.at[tb.at[c % 4]],
                              ssems.at[slot]).wait()

    def process(c, slot, cnt):
        def g_body(g, cnt):
            tv = tb[c % 4, pl.ds(g * L, L)]
            tb[c % 4, pl.ds(g * L, L)] = tv + region0
            for t in range(ET_NUM):
                p = plsc.all_reduce_population_count(tv == t)
                cnt = cnt + jnp.where(lane == t, p, 0)
            for rloc in range(L):
                row = g * L + rloc
                for f in range(F // L):
                    e = fb[slot, row, pl.ds(f * L, L)]
                    r = rb[slot, row, pl.ds(f * L, L)]
                    d = e - r
                    ab[slot, row, pl.ds(f * L, L)] = jnp.abs(d)
                    sb[slot, row, pl.ds(f * L, L)] = d * d
            return cnt

        return lax.fori_loop(0, GROUPS, g_body, cnt)

    dma_start(0, 0)
    dma_start(1, 1)

    def pair_body(cc, cnt):
        c0 = 2 * cc
        dma_wait(c0, 0)
        pl.when(cc >= 1)(lambda: scat_wait(c0 - 2, 0))
        cnt = process(c0, 0, cnt)
        scat_start(c0, 0)
        pl.when(c0 + 2 < nchunks)(lambda: dma_start(c0 + 2, 0))
        dma_wait(c0 + 1, 1)
        pl.when(cc >= 1)(lambda: scat_wait(c0 - 1, 1))
        cnt = process(c0 + 1, 1, cnt)
        scat_start(c0 + 1, 1)
        pl.when(c0 + 3 < nchunks)(lambda: dma_start(c0 + 3, 1))
        return cnt

    cnt = lax.fori_loop(0, nchunks // 2, pair_body, cnt0)
    last = nchunks - 1
    dma_wait(last, 0)
    scat_wait(last - 2, 0)
    cnt = process(last, 0, cnt)
    scat_start(last, 0)
    scat_wait(last, 0)
    scat_wait(last - 1, 1)
    return cnt


def _broadcast_counts(cnt_vec, cnt_vmem, cntb, rows):
    cnt_vmem[pl.ds(0, L)] = cnt_vec
    for r in range(rows):
        v = plsc.load_gather(cnt_vmem, [jnp.full((L,), r, jnp.int32)])
        for f in range(F // L):
            cntb[r, pl.ds(f * L, L)] = v


def _sc_partials(edge, ref_edge, et, node, ref_node, at):
    mesh = plsc.VectorSubcoreMesh(core_axis_name="c", subcore_axis_name="s")

    @functools.partial(
        pl.kernel,
        out_type=(
            jax.ShapeDtypeStruct((NW, ET_NUM, F), jnp.float32),
            jax.ShapeDtypeStruct((NW, ET_NUM, F), jnp.float32),
            jax.ShapeDtypeStruct((NW, ET_NUM, F), jnp.float32),
            jax.ShapeDtypeStruct((NW, AT_NUM, F), jnp.float32),
            jax.ShapeDtypeStruct((NW, AT_NUM, F), jnp.float32),
            jax.ShapeDtypeStruct((NW, AT_NUM, F), jnp.float32),
        ),
        mesh=mesh,
        compiler_params=pltpu.CompilerParams(needs_layout_passes=False),
        scratch_types=[
            pltpu.VMEM((2, CHUNK, F), jnp.float32),
            pltpu.VMEM((2, CHUNK, F), jnp.float32),
            pltpu.VMEM((4, CHUNK), jnp.int32),
            pltpu.VMEM((2, CHUNK, F), jnp.float32),
            pltpu.VMEM((2, CHUNK, F), jnp.float32),
            pltpu.VMEM_SHARED((L * ET_NUM, F), jnp.float32),
            pltpu.VMEM_SHARED((L * ET_NUM, F), jnp.float32),
            pltpu.VMEM_SHARED((L * 8, F), jnp.float32),
            pltpu.VMEM_SHARED((L * 8, F), jnp.float32),
            pltpu.VMEM((L,), jnp.float32),
            pltpu.VMEM((ET_NUM, F), jnp.float32),
            pltpu.SemaphoreType.DMA((2,)),
            pltpu.SemaphoreType.DMA((2,)),
        ],
    )
    def sc(edge_h, refe_h, et_h, node_h, refn_h, at_h,
           out_ea, out_es, out_ec, out_na, out_ns, out_nc,
           fb, rb, tb, ab, sb, acc_ea, acc_es, acc_na, acc_ns,
           cnt_vmem, cntb, dsems, ssems):
        sid = lax.axis_index("s")
        wid = sid * 2 + lax.axis_index("c")

        _zero_rows(cntb, ET_NUM)
        pltpu.sync_copy(cntb, acc_ea.at[pl.ds(sid * ET_NUM, ET_NUM)])
        pltpu.sync_copy(cntb, acc_es.at[pl.ds(sid * ET_NUM, ET_NUM)])
        pltpu.sync_copy(cntb.at[pl.ds(0, 8)], acc_na.at[pl.ds(sid * 8, 8)])
        pltpu.sync_copy(cntb.at[pl.ds(0, 8)], acc_ns.at[pl.ds(sid * 8, 8)])

        e_per_w = E_SC // NW
        n_per_w = N_PAD // NW

        cnt_e = _stream_accum(edge_h, refe_h, et_h,
                              wid * e_per_w, wid * e_per_w, e_per_w // CHUNK,
                              fb, rb, tb, ab, sb, dsems, ssems,
                              acc_ea, acc_es, sid * ET_NUM,
                              jnp.zeros((L,), jnp.int32))
        cnt_n = _stream_accum(node_h, refn_h, at_h,
                              wid * n_per_w, wid * n_per_w, n_per_w // CHUNK,
                              fb, rb, tb, ab, sb, dsems, ssems,
                              acc_na, acc_ns, sid * 8,
                              jnp.zeros((L,), jnp.int32))

        pltpu.sync_copy(acc_ea.at[pl.ds(sid * ET_NUM, ET_NUM)], out_ea.at[wid])
        pltpu.sync_copy(acc_es.at[pl.ds(sid * ET_NUM, ET_NUM)], out_es.at[wid])
        _broadcast_counts(cnt_e.astype(jnp.float32), cnt_vmem, cntb, ET_NUM)
        pltpu.sync_copy(cntb, out_ec.at[wid])
        pltpu.sync_copy(acc_na.at[pl.ds(sid * 8, AT_NUM)], out_na.at[wid])
        pltpu.sync_copy(acc_ns.at[pl.ds(sid * 8, AT_NUM)], out_ns.at[wid])
        _broadcast_counts(cnt_n.astype(jnp.float32), cnt_vmem, cntb, AT_NUM)
        pltpu.sync_copy(cntb.at[pl.ds(0, AT_NUM)], out_nc.at[wid])

    return sc(edge, ref_edge, et, node, ref_node, at)


def _tc_edge_kernel(ft, rt, ty, oa, os_, oc):
    i = pl.program_id(0)
    d = ft[...] - rt[...]
    a = jnp.abs(d)
    s = d * d
    onehot = (lax.broadcasted_iota(jnp.int32, (ET_NUM, TCB), 0)
              == ty[...]).astype(jnp.float32)
    dims = (((1,), (0,)), ((), ()))
    pa = lax.dot_general(onehot, a, dims, preferred_element_type=jnp.float32)
    ps = lax.dot_general(onehot, s, dims, preferred_element_type=jnp.float32)
    pc = lax.dot_general(onehot, jnp.ones((TCB, F), jnp.float32), dims,
                         preferred_element_type=jnp.float32)

    @pl.when(i == 0)
    def _init():
        oa[...] = jnp.zeros_like(oa)
        os_[...] = jnp.zeros_like(os_)
        oc[...] = jnp.zeros_like(oc)

    oa[...] += pa
    os_[...] += ps
    oc[...] += pc


def _tc_partials(edge, ref_edge, etf):
    nblocks = (E_ROWS - E_SC) // TCB
    blk0 = E_SC // TCB
    return pl.pallas_call(
        _tc_edge_kernel,
        grid=(nblocks,),
        in_specs=[
            pl.BlockSpec((TCB, F), lambda i: (blk0 + i, 0)),
            pl.BlockSpec((TCB, F), lambda i: (blk0 + i, 0)),
            pl.BlockSpec((1, TCB), lambda i: (0, blk0 + i)),
        ],
        out_specs=[
            pl.BlockSpec((ET_NUM, F), lambda i: (0, 0)),
            pl.BlockSpec((ET_NUM, F), lambda i: (0, 0)),
            pl.BlockSpec((ET_NUM, F), lambda i: (0, 0)),
        ],
        out_shape=[
            jax.ShapeDtypeStruct((ET_NUM, F), jnp.float32),
            jax.ShapeDtypeStruct((ET_NUM, F), jnp.float32),
            jax.ShapeDtypeStruct((ET_NUM, F), jnp.float32),
        ],
    )(edge, ref_edge, etf)


def _combine_kernel(ea, es, ec, na, ns, nc, ta, ts, tcn, nmask, emask, out):
    def half_loss(s_abs, s_sq, cnt, mask):
        present = (cnt > 0.0).astype(jnp.float32)
        sel = present * mask
        denom = jnp.maximum(cnt, 1.0)
        ncnt = jnp.sum(sel)
        term_abs = jnp.sum(sel * s_abs / denom) / ncnt
        term_sq = jnp.sum(sel * s_sq / denom) / ncnt
        return 0.5 * (term_abs + jnp.sqrt(term_sq))

    hop = half_loss(jnp.sum(ea[...], axis=0) + ta[...],
                    jnp.sum(es[...], axis=0) + ts[...],
                    jnp.sum(ec[...], axis=0) + tcn[...], emask[...])
    ons = half_loss(jnp.sum(na[...], axis=0), jnp.sum(ns[...], axis=0),
                    jnp.sum(nc[...], axis=0), nmask[...])
    out[0, 0] = 0.5 * (ons + hop)


def kernel(node_features, ref_node_features, edge_features, ref_edge_features,
           atom_type, edge_type, mask_to_nrme, mask_to_erme):
    at = atom_type.astype(jnp.int32)
    et = edge_type.astype(jnp.int32)
    n = node_features.shape[0]
    at_pad = jnp.concatenate([at, jnp.full((N_PAD - n,), AT_NUM, jnp.int32)])
    zpad = jnp.zeros((N_PAD - n, F), jnp.float32)
    nf = jnp.concatenate([node_features, zpad])
    rnf = jnp.concatenate([ref_node_features, zpad])

    ea, es, ec, na, ns, nc = _sc_partials(edge_features, ref_edge_features, et,
                                          nf, rnf, at_pad)
    etf = et.reshape(1, E_ROWS)
    ta, ts, tcn = _tc_partials(edge_features, ref_edge_features, etf)

    loss = pl.pallas_call(
        _combine_kernel,
        out_shape=jax.ShapeDtypeStruct((1, 1), jnp.float32),
        out_specs=pl.BlockSpec(memory_space=pltpu.SMEM),
    )(ea, es, ec, na, ns, nc, ta, ts, tcn,
      mask_to_nrme.astype(jnp.float32), mask_to_erme.astype(jnp.float32))
    return loss[0, 0]
